# speedup vs baseline: 2.8709x; 2.8709x over previous
import functools

import jax
import jax.numpy as jnp
from jax import lax
from jax.experimental import pallas as pl
from jax.experimental.pallas import tpu as pltpu
from jax.experimental.pallas import tpu_sc as plsc

B, L, V, D, H = 16384, 50, 1000000, 128, 32
NC, NS = 2, 16
NW = NC * NS
GP = 2
IPG = GP * L
NG = 2 * B // GP
GPW = NG // NW
BLK = 32
NBLK = GPW // BLK
ROWS_BLK = BLK * GP
LANES = D // 16


def _pool_sc(xs, embed):
    mesh = plsc.VectorSubcoreMesh(core_axis_name="c", subcore_axis_name="s")

    @functools.partial(
        pl.kernel,
        out_type=jax.ShapeDtypeStruct((2 * B, D), jnp.float32),
        mesh=mesh,
        scratch_types=[
            pltpu.VMEM((BLK, IPG), jnp.int32),
            pltpu.VMEM((IPG, D), jnp.float32),
            pltpu.VMEM((ROWS_BLK, D), jnp.float32),
            pltpu.SemaphoreType.DMA,
        ],
    )
    def pool(xs_hbm, emb_hbm, out_hbm, idx_v, rows_v, out_v, sem):
        wid = lax.axis_index("s") * NC + lax.axis_index("c")
        g0 = wid * GPW

        def blk_body(b, carry):
            gbase = g0 + b * BLK
            pltpu.sync_copy(xs_hbm.at[pl.ds(gbase, BLK)], idx_v)

            def grp_body(k, carry2):
                pltpu.async_copy(emb_hbm.at[idx_v.at[k]], rows_v, sem).wait()

                def pool_row(slot):
                    base = slot * L

                    def racc(r, accs):
                        return tuple(
                            accs[c] + rows_v[base + r, pl.ds(c * 16, 16)]
                            for c in range(LANES)
                        )

                    accs = lax.fori_loop(
                        0, L, racc,
                        tuple(jnp.zeros((16,), jnp.float32) for _ in range(LANES)),
                    )
                    for c in range(LANES):
                        out_v[k * GP + slot, pl.ds(c * 16, 16)] = accs[c]

                pool_row(0)
                pool_row(1)
                return carry2

            lax.fori_loop(0, BLK, grp_body, 0)
            pltpu.sync_copy(out_v, out_hbm.at[pl.ds(gbase * GP, ROWS_BLK)])
            return carry

        lax.fori_loop(0, NBLK, blk_body, 0)

    return pool(xs, embed)


BT = 2048


def _mlp_tc(p1, p2, w1t, b1v, b1h, w2t, b2v, b2h):
    def body(p1_ref, p2_ref, w1_ref, b1v_ref, b1h_ref, w2_ref, b2v_ref,
             b2h_ref, o_ref):
        h1 = jnp.tanh(p1_ref[...] + b1v_ref[...])
        a1 = jnp.tanh(
            jnp.dot(h1, w1_ref[...], preferred_element_type=jnp.float32)
            + b1h_ref[...])
        h2 = jnp.tanh(p2_ref[...] + b2v_ref[...])
        a2 = jnp.tanh(
            jnp.dot(h2, w2_ref[...], preferred_element_type=jnp.float32)
            + b2h_ref[...])
        s = jnp.sum(a1 * a2, axis=1)
        o_ref[...] = (1.0 / (1.0 + jnp.exp(-s)))[None, :]

    return pl.pallas_call(
        body,
        grid=(B // BT,),
        in_specs=[
            pl.BlockSpec((BT, D), lambda i: (i, 0)),
            pl.BlockSpec((BT, D), lambda i: (i, 0)),
            pl.BlockSpec((D, H), lambda i: (0, 0)),
            pl.BlockSpec((1, D), lambda i: (0, 0)),
            pl.BlockSpec((1, H), lambda i: (0, 0)),
            pl.BlockSpec((D, H), lambda i: (0, 0)),
            pl.BlockSpec((1, D), lambda i: (0, 0)),
            pl.BlockSpec((1, H), lambda i: (0, 0)),
        ],
        out_specs=pl.BlockSpec((1, BT), lambda i: (0, i)),
        out_shape=jax.ShapeDtypeStruct((1, B), jnp.float32),
    )(p1, p2, w1t, b1v, b1h, w2t, b2v, b2h)


def kernel(x1, x2, embed, t1_bias1, t1_W, t1_b, t2_bias1, t2_W, t2_b):
    xs = jnp.concatenate([x1, x2], axis=0).astype(jnp.int32).reshape(NG, IPG)
    pooled = _pool_sc(xs, embed)
    out = _mlp_tc(
        pooled[:B], pooled[B:],
        t1_W.T, t1_bias1[None, :], t1_b[None, :],
        t2_W.T, t2_bias1[None, :], t2_b[None, :],
    )
    return out.reshape(B)


# speedup vs baseline: 5.0867x; 1.7718x over previous
import functools

import jax
import jax.numpy as jnp
from jax import lax
from jax.experimental import pallas as pl
from jax.experimental.pallas import tpu as pltpu
from jax.experimental.pallas import tpu_sc as plsc

B, L, V, D, H = 16384, 50, 1000000, 128, 32
NC, NS = 2, 16
NW = NC * NS
GP = 2
IPG = GP * L
NG = 2 * B // GP
GPW = NG // NW
BLK = 32
NBLK = GPW // BLK
ROWS_BLK = BLK * GP
LANES = D // 16


UNROLL = 5


def _pool_sc(xs, embed):
    mesh = plsc.VectorSubcoreMesh(core_axis_name="c", subcore_axis_name="s")

    @functools.partial(
        pl.kernel,
        out_type=jax.ShapeDtypeStruct((2 * B, D), jnp.float32),
        mesh=mesh,
        scratch_types=[
            pltpu.VMEM((GPW, IPG), jnp.int32),
            pltpu.VMEM((IPG, D), jnp.float32),
            pltpu.VMEM((IPG, D), jnp.float32),
            pltpu.VMEM((ROWS_BLK, D), jnp.float32),
            pltpu.SemaphoreType.DMA,
            pltpu.SemaphoreType.DMA,
        ],
    )
    def pool(xs_hbm, emb_hbm, out_hbm, idx_v, rows0_v, rows1_v, out_v,
             sem0, sem1):
        wid = lax.axis_index("s") * NC + lax.axis_index("c")
        g0 = wid * GPW
        pltpu.sync_copy(xs_hbm.at[pl.ds(g0, GPW)], idx_v)

        def start(k, buf, sem):
            pltpu.async_copy(emb_hbm.at[idx_v.at[k]], buf, sem)

        def accum(buf, out_row):
            for slot in range(GP):
                base = slot * L

                def racc(r, accs):
                    for dr in range(UNROLL):
                        accs = tuple(
                            accs[c] + buf[base + r * UNROLL + dr,
                                          pl.ds(c * 16, 16)]
                            for c in range(LANES)
                        )
                    return accs

                accs = lax.fori_loop(
                    0, L // UNROLL, racc,
                    tuple(jnp.zeros((16,), jnp.float32) for _ in range(LANES)),
                )
                for c in range(LANES):
                    out_v[out_row + slot, pl.ds(c * 16, 16)] = accs[c]

        start(0, rows0_v, sem0)

        def step(k2, carry):
            k = k2 * 2
            m = lax.rem(k2, NBLK)
            blk = lax.div(k2, NBLK)
            start(k + 1, rows1_v, sem1)
            pltpu.make_async_copy(emb_hbm.at[idx_v.at[0]], rows0_v, sem0).wait()
            accum(rows0_v, m * 2 * GP)
            start(jnp.minimum(k + 2, GPW - 1), rows0_v, sem0)
            pltpu.make_async_copy(emb_hbm.at[idx_v.at[0]], rows1_v, sem1).wait()
            accum(rows1_v, m * 2 * GP + GP)

            @pl.when(m == NBLK - 1)
            def _flush():
                pltpu.sync_copy(
                    out_v,
                    out_hbm.at[pl.ds((g0 + blk * BLK) * GP, ROWS_BLK)])

            return carry

        lax.fori_loop(0, GPW // 2, step, 0)
        pltpu.make_async_copy(emb_hbm.at[idx_v.at[0]], rows0_v, sem0).wait()

    return pool(xs, embed)


BT = 2048


def _mlp_tc(p1, p2, w1t, b1v, b1h, w2t, b2v, b2h):
    def body(p1_ref, p2_ref, w1_ref, b1v_ref, b1h_ref, w2_ref, b2v_ref,
             b2h_ref, o_ref):
        h1 = jnp.tanh(p1_ref[...] + b1v_ref[...])
        a1 = jnp.tanh(
            jnp.dot(h1, w1_ref[...], preferred_element_type=jnp.float32)
            + b1h_ref[...])
        h2 = jnp.tanh(p2_ref[...] + b2v_ref[...])
        a2 = jnp.tanh(
            jnp.dot(h2, w2_ref[...], preferred_element_type=jnp.float32)
            + b2h_ref[...])
        s = jnp.sum(a1 * a2, axis=1)
        o_ref[...] = (1.0 / (1.0 + jnp.exp(-s)))[None, :]

    return pl.pallas_call(
        body,
        grid=(B // BT,),
        in_specs=[
            pl.BlockSpec((BT, D), lambda i: (i, 0)),
            pl.BlockSpec((BT, D), lambda i: (i, 0)),
            pl.BlockSpec((D, H), lambda i: (0, 0)),
            pl.BlockSpec((1, D), lambda i: (0, 0)),
            pl.BlockSpec((1, H), lambda i: (0, 0)),
            pl.BlockSpec((D, H), lambda i: (0, 0)),
            pl.BlockSpec((1, D), lambda i: (0, 0)),
            pl.BlockSpec((1, H), lambda i: (0, 0)),
        ],
        out_specs=pl.BlockSpec((1, BT), lambda i: (0, i)),
        out_shape=jax.ShapeDtypeStruct((1, B), jnp.float32),
    )(p1, p2, w1t, b1v, b1h, w2t, b2v, b2h)


def kernel(x1, x2, embed, t1_bias1, t1_W, t1_b, t2_bias1, t2_W, t2_b):
    xs = jnp.concatenate([x1, x2], axis=0).astype(jnp.int32).reshape(NG, IPG)
    pooled = _pool_sc(xs, embed)
    out = _mlp_tc(
        pooled[:B], pooled[B:],
        t1_W.T, t1_bias1[None, :], t1_b[None, :],
        t2_W.T, t2_bias1[None, :], t2_b[None, :],
    )
    return out.reshape(B)


# speedup vs baseline: 5.1852x; 1.0194x over previous
import functools

import jax
import jax.numpy as jnp
from jax import lax
from jax.experimental import pallas as pl
from jax.experimental.pallas import tpu as pltpu
from jax.experimental.pallas import tpu_sc as plsc

B, L, V, D, H = 16384, 50, 1000000, 128, 32
NC, NS = 2, 16
NW = NC * NS
GP = 2
IPG = GP * L
NG = B // GP
GPW = NG // NW
BLK = 32
NBLK = BLK // 2
ROWS_BLK = BLK * GP
LANES = D // 16


UNROLL = 5


def _pool_sc(xs1, xs2, embed):
    mesh = plsc.VectorSubcoreMesh(core_axis_name="c", subcore_axis_name="s")

    @functools.partial(
        pl.kernel,
        out_type=jax.ShapeDtypeStruct((2 * B, D), jnp.float32),
        mesh=mesh,
        scratch_types=[
            pltpu.VMEM((2 * GPW, IPG), jnp.int32),
            pltpu.VMEM((IPG, D), jnp.float32),
            pltpu.VMEM((IPG, D), jnp.float32),
            pltpu.VMEM((ROWS_BLK, D), jnp.float32),
            pltpu.SemaphoreType.DMA,
            pltpu.SemaphoreType.DMA,
        ],
    )
    def pool(xs1_hbm, xs2_hbm, emb_hbm, out_hbm, idx_v, rows0_v, rows1_v,
             out_v, sem0, sem1):
        wid = lax.axis_index("s") * NC + lax.axis_index("c")
        g0 = wid * GPW
        pltpu.sync_copy(xs1_hbm.at[pl.ds(g0, GPW)], idx_v.at[pl.ds(0, GPW)])
        pltpu.sync_copy(xs2_hbm.at[pl.ds(g0, GPW)], idx_v.at[pl.ds(GPW, GPW)])

        def start(k, buf, sem):
            pltpu.async_copy(emb_hbm.at[idx_v.at[k]], buf, sem)

        def accum(buf, out_row):
            for slot in range(GP):
                base = slot * L

                def racc(r, accs):
                    for dr in range(UNROLL):
                        accs = tuple(
                            accs[c] + buf[base + r * UNROLL + dr,
                                          pl.ds(c * 16, 16)]
                            for c in range(LANES)
                        )
                    return accs

                accs = lax.fori_loop(
                    0, L // UNROLL, racc,
                    tuple(jnp.zeros((16,), jnp.float32) for _ in range(LANES)),
                )
                for c in range(LANES):
                    out_v[out_row + slot, pl.ds(c * 16, 16)] = accs[c]

        def run_tower(tower):
            kofs = tower * GPW
            out0 = tower * B + g0 * GP
            start(kofs, rows0_v, sem0)

            def step(k2, carry):
                k = kofs + k2 * 2
                m = lax.rem(k2, NBLK)
                blk = lax.div(k2, NBLK)
                start(k + 1, rows1_v, sem1)
                pltpu.make_async_copy(
                    emb_hbm.at[idx_v.at[0]], rows0_v, sem0).wait()
                accum(rows0_v, m * 2 * GP)
                start(jnp.minimum(k + 2, kofs + GPW - 1), rows0_v, sem0)
                pltpu.make_async_copy(
                    emb_hbm.at[idx_v.at[0]], rows1_v, sem1).wait()
                accum(rows1_v, m * 2 * GP + GP)

                @pl.when(m == NBLK - 1)
                def _flush():
                    pltpu.sync_copy(
                        out_v,
                        out_hbm.at[pl.ds(out0 + blk * ROWS_BLK, ROWS_BLK)])

                return carry

            lax.fori_loop(0, GPW // 2, step, 0)
            pltpu.make_async_copy(emb_hbm.at[idx_v.at[0]], rows0_v, sem0).wait()

        run_tower(0)
        run_tower(1)

    return pool(xs1, xs2, embed)


BT = 2048


def _mlp_tc(pooled, w1t, b1v, b1h, w2t, b2v, b2h):
    def body(p1_ref, p2_ref, w1_ref, b1v_ref, b1h_ref, w2_ref, b2v_ref,
             b2h_ref, o_ref):
        h1 = jnp.tanh(p1_ref[...] + b1v_ref[...])
        a1 = jnp.tanh(
            jnp.dot(h1, w1_ref[...], preferred_element_type=jnp.float32)
            + b1h_ref[...])
        h2 = jnp.tanh(p2_ref[...] + b2v_ref[...])
        a2 = jnp.tanh(
            jnp.dot(h2, w2_ref[...], preferred_element_type=jnp.float32)
            + b2h_ref[...])
        s = jnp.sum(a1 * a2, axis=1)
        o_ref[...] = (1.0 / (1.0 + jnp.exp(-s)))[None, :]

    return pl.pallas_call(
        body,
        grid=(B // BT,),
        in_specs=[
            pl.BlockSpec((BT, D), lambda i: (i, 0)),
            pl.BlockSpec((BT, D), lambda i: (i + B // BT, 0)),
            pl.BlockSpec((D, H), lambda i: (0, 0)),
            pl.BlockSpec((1, D), lambda i: (0, 0)),
            pl.BlockSpec((1, H), lambda i: (0, 0)),
            pl.BlockSpec((D, H), lambda i: (0, 0)),
            pl.BlockSpec((1, D), lambda i: (0, 0)),
            pl.BlockSpec((1, H), lambda i: (0, 0)),
        ],
        out_specs=pl.BlockSpec((1, BT), lambda i: (0, i)),
        out_shape=jax.ShapeDtypeStruct((1, B), jnp.float32),
    )(pooled, pooled, w1t, b1v, b1h, w2t, b2v, b2h)


def kernel(x1, x2, embed, t1_bias1, t1_W, t1_b, t2_bias1, t2_W, t2_b):
    xs1 = x1.astype(jnp.int32).reshape(NG, IPG)
    xs2 = x2.astype(jnp.int32).reshape(NG, IPG)
    pooled = _pool_sc(xs1, xs2, embed)
    out = _mlp_tc(
        pooled,
        t1_W.T, t1_bias1[None, :], t1_b[None, :],
        t2_W.T, t2_bias1[None, :], t2_b[None, :],
    )
    return out.reshape(B)


# speedup vs baseline: 7.5058x; 1.4475x over previous
import functools

import jax
import jax.numpy as jnp
from jax import lax
from jax.experimental import pallas as pl
from jax.experimental.pallas import tpu as pltpu
from jax.experimental.pallas import tpu_sc as plsc

B, L, V, D, H = 16384, 50, 1000000, 128, 32
NC, NS = 2, 16
NW = NC * NS
GP = 2
IPG = GP * L
NG = B // GP
GPW = NG // NW
BLK = 32
NBLK = BLK // 2
ROWS_BLK = BLK * GP
LANES = D // 16


UNROLL = 5
NBUF = 4


def _pool_sc(xs1, xs2, embed):
    mesh = plsc.VectorSubcoreMesh(core_axis_name="c", subcore_axis_name="s")

    @functools.partial(
        pl.kernel,
        out_type=jax.ShapeDtypeStruct((2 * B, D), jnp.float32),
        mesh=mesh,
        scratch_types=[
            pltpu.VMEM((2 * GPW, IPG), jnp.int32),
            [pltpu.VMEM((IPG, D), jnp.float32) for _ in range(NBUF)],
            pltpu.VMEM((ROWS_BLK, D), jnp.float32),
            [pltpu.SemaphoreType.DMA for _ in range(NBUF)],
        ],
    )
    def pool(xs1_hbm, xs2_hbm, emb_hbm, out_hbm, idx_v, rows_bufs, out_v,
             sems):
        wid = lax.axis_index("s") * NC + lax.axis_index("c")
        g0 = wid * GPW
        pltpu.sync_copy(xs1_hbm.at[pl.ds(g0, GPW)], idx_v.at[pl.ds(0, GPW)])
        pltpu.sync_copy(xs2_hbm.at[pl.ds(g0, GPW)], idx_v.at[pl.ds(GPW, GPW)])

        def start(k, j):
            pltpu.async_copy(emb_hbm.at[idx_v.at[k]], rows_bufs[j], sems[j])

        def wait(j):
            pltpu.make_async_copy(
                emb_hbm.at[idx_v.at[0]], rows_bufs[j], sems[j]).wait()

        def accum(buf, out_row):
            for slot in range(GP):
                base = slot * L

                def racc(r, accs):
                    for dr in range(UNROLL):
                        accs = tuple(
                            accs[c] + buf[base + r * UNROLL + dr,
                                          pl.ds(c * 16, 16)]
                            for c in range(LANES)
                        )
                    return accs

                accs = lax.fori_loop(
                    0, L // UNROLL, racc,
                    tuple(jnp.zeros((16,), jnp.float32) for _ in range(LANES)),
                )
                for c in range(LANES):
                    out_v[out_row + slot, pl.ds(c * 16, 16)] = accs[c]

        def run_tower(tower):
            kofs = tower * GPW
            out0 = tower * B + g0 * GP
            for j in range(NBUF):
                start(kofs + j, j)

            def step(k4, carry):
                k = kofs + k4 * NBUF
                m = lax.rem(k4, BLK // NBUF)
                blk = lax.div(k4, BLK // NBUF)
                for j in range(NBUF):
                    wait(j)
                    accum(rows_bufs[j], (m * NBUF + j) * GP)
                    start(jnp.minimum(k + j + NBUF, kofs + GPW - 1), j)

                @pl.when(m == BLK // NBUF - 1)
                def _flush():
                    pltpu.sync_copy(
                        out_v,
                        out_hbm.at[pl.ds(out0 + blk * ROWS_BLK, ROWS_BLK)])

                return carry

            lax.fori_loop(0, GPW // NBUF, step, 0)
            for j in range(NBUF):
                wait(j)

        run_tower(0)
        run_tower(1)

    return pool(xs1, xs2, embed)


BT = 2048


def _mlp_tc(pooled, w1t, b1v, b1h, w2t, b2v, b2h):
    def body(p1_ref, p2_ref, w1_ref, b1v_ref, b1h_ref, w2_ref, b2v_ref,
             b2h_ref, o_ref):
        h1 = jnp.tanh(p1_ref[...] + b1v_ref[...])
        a1 = jnp.tanh(
            jnp.dot(h1, w1_ref[...], preferred_element_type=jnp.float32)
            + b1h_ref[...])
        h2 = jnp.tanh(p2_ref[...] + b2v_ref[...])
        a2 = jnp.tanh(
            jnp.dot(h2, w2_ref[...], preferred_element_type=jnp.float32)
            + b2h_ref[...])
        s = jnp.sum(a1 * a2, axis=1)
        o_ref[...] = (1.0 / (1.0 + jnp.exp(-s)))[None, :]

    return pl.pallas_call(
        body,
        grid=(B // BT,),
        in_specs=[
            pl.BlockSpec((BT, D), lambda i: (i, 0)),
            pl.BlockSpec((BT, D), lambda i: (i + B // BT, 0)),
            pl.BlockSpec((D, H), lambda i: (0, 0)),
            pl.BlockSpec((1, D), lambda i: (0, 0)),
            pl.BlockSpec((1, H), lambda i: (0, 0)),
            pl.BlockSpec((D, H), lambda i: (0, 0)),
            pl.BlockSpec((1, D), lambda i: (0, 0)),
            pl.BlockSpec((1, H), lambda i: (0, 0)),
        ],
        out_specs=pl.BlockSpec((1, BT), lambda i: (0, i)),
        out_shape=jax.ShapeDtypeStruct((1, B), jnp.float32),
    )(pooled, pooled, w1t, b1v, b1h, w2t, b2v, b2h)


def kernel(x1, x2, embed, t1_bias1, t1_W, t1_b, t2_bias1, t2_W, t2_b):
    xs1 = x1.astype(jnp.int32).reshape(NG, IPG)
    xs2 = x2.astype(jnp.int32).reshape(NG, IPG)
    pooled = _pool_sc(xs1, xs2, embed)
    out = _mlp_tc(
        pooled,
        t1_W.T, t1_bias1[None, :], t1_b[None, :],
        t2_W.T, t2_bias1[None, :], t2_b[None, :],
    )
    return out.reshape(B)


# speedup vs baseline: 7.9256x; 1.0559x over previous
import functools

import jax
import jax.numpy as jnp
from jax import lax
from jax.experimental import pallas as pl
from jax.experimental.pallas import tpu as pltpu
from jax.experimental.pallas import tpu_sc as plsc

B, L, V, D, H = 16384, 50, 1000000, 128, 32
NC, NS = 2, 16
NW = NC * NS
IPG = L
GPW = B // NW
BLK = 64
ROWS_BLK = BLK
LANES = D // 16


UNROLL = 10
NBUF = 8


def _pool_sc(xs1, xs2, embed):
    mesh = plsc.VectorSubcoreMesh(core_axis_name="c", subcore_axis_name="s")

    @functools.partial(
        pl.kernel,
        out_type=jax.ShapeDtypeStruct((2 * B, D), jnp.float32),
        mesh=mesh,
        scratch_types=[
            pltpu.VMEM((GPW, IPG), jnp.int32),
            [pltpu.VMEM((IPG, D), jnp.float32) for _ in range(NBUF)],
            pltpu.VMEM((ROWS_BLK, D), jnp.float32),
            [pltpu.SemaphoreType.DMA for _ in range(NBUF)],
        ],
    )
    def pool(xs1_hbm, xs2_hbm, emb_hbm, out_hbm, idx_v, rows_bufs, out_v,
             sems):
        wid = lax.axis_index("s") * NC + lax.axis_index("c")
        g0 = wid * GPW

        def start(k, j):
            pltpu.async_copy(emb_hbm.at[idx_v.at[k]], rows_bufs[j], sems[j])

        def wait(j):
            pltpu.make_async_copy(
                emb_hbm.at[idx_v.at[0]], rows_bufs[j], sems[j]).wait()

        def accum(buf, out_row):

            def racc(r, accs):
                for dr in range(UNROLL):
                    accs = tuple(
                        accs[c] + buf[r * UNROLL + dr, pl.ds(c * 16, 16)]
                        for c in range(LANES)
                    )
                return accs

            accs = lax.fori_loop(
                0, L // UNROLL, racc,
                tuple(jnp.zeros((16,), jnp.float32) for _ in range(LANES)),
            )
            for c in range(LANES):
                out_v[out_row, pl.ds(c * 16, 16)] = accs[c]

        def run_tower(xs_hbm, tower):
            out0 = tower * B + g0
            pltpu.sync_copy(xs_hbm.at[pl.ds(g0, GPW)], idx_v)
            for j in range(NBUF):
                start(j, j)

            def step(k4, carry):
                k = k4 * NBUF
                m = lax.rem(k4, BLK // NBUF)
                blk = lax.div(k4, BLK // NBUF)
                for j in range(NBUF):
                    wait(j)
                    accum(rows_bufs[j], m * NBUF + j)
                    start(jnp.minimum(k + j + NBUF, GPW - 1), j)

                @pl.when(m == BLK // NBUF - 1)
                def _flush():
                    pltpu.sync_copy(
                        out_v,
                        out_hbm.at[pl.ds(out0 + blk * ROWS_BLK, ROWS_BLK)])

                return carry

            lax.fori_loop(0, GPW // NBUF, step, 0)
            for j in range(NBUF):
                wait(j)

        run_tower(xs1_hbm, 0)
        run_tower(xs2_hbm, 1)

    return pool(xs1, xs2, embed)


BT = 2048


def _mlp_tc(pooled, w1t, b1v, b1h, w2t, b2v, b2h):
    def body(p1_ref, p2_ref, w1_ref, b1v_ref, b1h_ref, w2_ref, b2v_ref,
             b2h_ref, o_ref):
        h1 = jnp.tanh(p1_ref[...] + b1v_ref[...])
        a1 = jnp.tanh(
            jnp.dot(h1, w1_ref[...], preferred_element_type=jnp.float32)
            + b1h_ref[...])
        h2 = jnp.tanh(p2_ref[...] + b2v_ref[...])
        a2 = jnp.tanh(
            jnp.dot(h2, w2_ref[...], preferred_element_type=jnp.float32)
            + b2h_ref[...])
        s = jnp.sum(a1 * a2, axis=1)
        o_ref[...] = (1.0 / (1.0 + jnp.exp(-s)))[None, :]

    return pl.pallas_call(
        body,
        grid=(B // BT,),
        in_specs=[
            pl.BlockSpec((BT, D), lambda i: (i, 0)),
            pl.BlockSpec((BT, D), lambda i: (i + B // BT, 0)),
            pl.BlockSpec((D, H), lambda i: (0, 0)),
            pl.BlockSpec((1, D), lambda i: (0, 0)),
            pl.BlockSpec((1, H), lambda i: (0, 0)),
            pl.BlockSpec((D, H), lambda i: (0, 0)),
            pl.BlockSpec((1, D), lambda i: (0, 0)),
            pl.BlockSpec((1, H), lambda i: (0, 0)),
        ],
        out_specs=pl.BlockSpec((1, BT), lambda i: (0, i)),
        out_shape=jax.ShapeDtypeStruct((1, B), jnp.float32),
    )(pooled, pooled, w1t, b1v, b1h, w2t, b2v, b2h)


def kernel(x1, x2, embed, t1_bias1, t1_W, t1_b, t2_bias1, t2_W, t2_b):
    pooled = _pool_sc(x1.astype(jnp.int32), x2.astype(jnp.int32), embed)
    out = _mlp_tc(
        pooled,
        t1_W.T, t1_bias1[None, :], t1_b[None, :],
        t2_W.T, t2_bias1[None, :], t2_b[None, :],
    )
    return out.reshape(B)
